# Initial kernel scaffold; baseline (speedup 1.0000x reference)
import jax
import jax.numpy as jnp
from jax.experimental import pallas as pl


def kernel(tokens, table):
    raise NotImplementedError("write your pallas kernel here")


# speedup vs baseline: 2.5105x; 2.5105x over previous
import functools

import jax
import jax.numpy as jnp
from jax import lax
from jax.experimental import pallas as pl
from jax.experimental.pallas import tpu as pltpu, tpu_sc as plsc

NUM_CHARS = 1000
D = 16
L_TOK = 20
ROWS_TOK = 22
NT = 64 * 256
NC, NS, LANES = 2, 16, 16
NW = NC * NS
T_PER_W = NT // NW
R_PER_W = T_PER_W * ROWS_TOK
CH_T = 64
CH_R = CH_T * ROWS_TOK
N_CH = T_PER_W // CH_T
G_ROWS = 128

_mesh = plsc.VectorSubcoreMesh(core_axis_name="c", subcore_axis_name="s")


@functools.partial(
    pl.kernel,
    out_type=jax.ShapeDtypeStruct((NT * ROWS_TOK, D), jnp.float32),
    mesh=_mesh,
    compiler_params=pltpu.CompilerParams(
        use_tc_tiling_on_sc=False, needs_layout_passes=False),
    scratch_types=[
        pltpu.VMEM((T_PER_W * L_TOK,), jnp.int32),
        pltpu.VMEM((CH_R,), jnp.int32),
        pltpu.VMEM((CH_R, D), jnp.float32),
        pltpu.SemaphoreType.DMA,
    ],
)
def _emb_lookup(tok_hbm, table_hbm, out_hbm, tok_v, fidx_v, rows_v, sem):
    w = lax.axis_index("s") * NC + lax.axis_index("c")
    tok_base = pl.multiple_of(w * (T_PER_W * L_TOK), T_PER_W * L_TOK)
    out_base = pl.multiple_of(w * R_PER_W, R_PER_W)
    pltpu.sync_copy(tok_hbm.at[pl.ds(tok_base, T_PER_W * L_TOK)], tok_v)

    def chunk(c, carry):
        base_r = c * CH_R

        def build(v, carry2):
            base = jnp.full((LANES,), base_r + v * LANES, jnp.int32)
            r = base + lax.iota(jnp.int32, LANES)
            q = lax.div(r, jnp.full((LANES,), ROWS_TOK, jnp.int32))
            l = r - q * jnp.full((LANES,), ROWS_TOK, jnp.int32)
            src = q * jnp.full((LANES,), L_TOK, jnp.int32) + l
            src = src - jnp.full((LANES,), 1, jnp.int32)
            src = jnp.maximum(src, jnp.full((LANES,), 0, jnp.int32))
            src = jnp.minimum(src, jnp.full((LANES,), T_PER_W * L_TOK - 1, jnp.int32))
            tv = plsc.load_gather(tok_v, [src])
            tv = jnp.where(l == jnp.full((LANES,), 0, jnp.int32),
                           jnp.full((LANES,), NUM_CHARS, jnp.int32), tv)
            tv = jnp.where(l == jnp.full((LANES,), ROWS_TOK - 1, jnp.int32),
                           jnp.full((LANES,), NUM_CHARS + 1, jnp.int32), tv)
            fidx_v[pl.ds(v * LANES, LANES)] = tv
            return carry2

        lax.fori_loop(0, CH_R // LANES, build, 0)

        for g in range(CH_R // G_ROWS):
            pltpu.async_copy(
                table_hbm.at[fidx_v.at[pl.ds(g * G_ROWS, G_ROWS)]],
                rows_v.at[pl.ds(g * G_ROWS, G_ROWS)],
                sem,
            ).wait()
        pltpu.sync_copy(rows_v, out_hbm.at[pl.ds(out_base + base_r, CH_R)])
        return carry

    lax.fori_loop(0, N_CH, chunk, 0)


def kernel(tokens, table):
    B1, B2, L = tokens.shape
    tok_flat = tokens.reshape(B1 * B2 * L).astype(jnp.int32)
    out = _emb_lookup(tok_flat, table)
    return out.reshape(B1, B2, ROWS_TOK, D)


# speedup vs baseline: 3.6586x; 1.4573x over previous
import functools

import jax
import jax.numpy as jnp
from jax import lax
from jax.experimental import pallas as pl
from jax.experimental.pallas import tpu as pltpu, tpu_sc as plsc

NUM_CHARS = 1000
D = 16
L_TOK = 20
ROWS_TOK = 22
B1, B2 = 64, 256
NT = B1 * B2
NC, NS, LANES = 2, 16, 16
NW = NC * NS
B1_PER_W = B1 // NW
T_PER_W = NT // NW
CH_T = 128
CH_R = CH_T * ROWS_TOK
N_CH = T_PER_W // CH_T
NBUF = 2
FIRE_U = 4

_mesh = plsc.VectorSubcoreMesh(core_axis_name="c", subcore_axis_name="s")


@functools.partial(
    pl.kernel,
    out_type=jax.ShapeDtypeStruct((B1, B2, ROWS_TOK, D), jnp.float32),
    mesh=_mesh,
    compiler_params=pltpu.CompilerParams(
        use_tc_tiling_on_sc=False, needs_layout_passes=False),
    scratch_types=[
        pltpu.VMEM((B1_PER_W, B2, L_TOK), jnp.int32),
        pltpu.VMEM((NBUF, CH_T, ROWS_TOK), jnp.int32),
        pltpu.VMEM((NBUF, CH_T, ROWS_TOK, D), jnp.float32),
        pltpu.SemaphoreType.DMA,
        pltpu.SemaphoreType.DMA,
    ],
)
def _emb_lookup(tok_hbm, table_hbm, out_hbm, tok_v, fidx_v, rows_v, gsem, wsem):
    w = lax.axis_index("s") * NC + lax.axis_index("c")
    b1_base = w * B1_PER_W
    pltpu.sync_copy(tok_hbm.at[pl.ds(b1_base, B1_PER_W)], tok_v)

    c22 = jnp.full((LANES,), ROWS_TOK, jnp.int32)
    c20 = jnp.full((LANES,), L_TOK, jnp.int32)
    cb2 = jnp.full((LANES,), B2 * L_TOK, jnp.int32)
    c1 = jnp.full((LANES,), 1, jnp.int32)
    c0 = jnp.full((LANES,), 0, jnp.int32)
    c21 = jnp.full((LANES,), ROWS_TOK - 1, jnp.int32)
    cmax = jnp.full((LANES,), T_PER_W * L_TOK - 1, jnp.int32)
    cstart = jnp.full((LANES,), NUM_CHARS, jnp.int32)
    cend = jnp.full((LANES,), NUM_CHARS + 1, jnp.int32)
    lanes_iota = lax.iota(jnp.int32, LANES)

    UNROLL = 4

    def build_chunk(c, b):
        base_r = c * CH_R
        ctok0 = jnp.full((LANES,), c * CH_T, jnp.int32)
        fidx_b = fidx_v.at[b]

        def build(v, carry2):
            for u in range(UNROLL):
                base = jnp.full(
                    (LANES,), base_r + (v * UNROLL + u) * LANES, jnp.int32)
                r = base + lanes_iota
                q = lax.div(r, c22)
                l = r - q * c22
                src = q * c20 + l - c1
                src = jnp.minimum(jnp.maximum(src, c0), cmax)
                i2 = lax.div(src, cb2)
                rem = src - i2 * cb2
                j = lax.div(rem, c20)
                k = rem - j * c20
                tv = plsc.load_gather(tok_v, [i2, j, k])
                tv = jnp.where(l == c0, cstart, tv)
                tv = jnp.where(l == c21, cend, tv)
                plsc.store_scatter(fidx_b, [q - ctok0, l], tv)
            return carry2

        lax.fori_loop(0, CH_R // (LANES * UNROLL), build, 0)

    def fire_gathers(b):
        def fire(g, carry):
            for u in range(FIRE_U):
                t = g * FIRE_U + u
                pltpu.async_copy(
                    table_hbm.at[fidx_v.at[b, t]],
                    rows_v.at[b, t],
                    gsem,
                )
            return carry

        lax.fori_loop(0, CH_T // FIRE_U, fire, 0)

    def drain_gathers(b, b1, t0):
        pltpu.make_async_copy(
            out_hbm.at[b1, pl.ds(t0, CH_T)], rows_v.at[b], gsem).wait()

    write_handles = [None] * N_CH
    build_chunk(0, 0)
    fire_gathers(0)
    for c in range(N_CH):
        b = c % NBUF
        nb = (c + 1) % NBUF
        b1 = b1_base + c // 2
        t0 = (c % 2) * CH_T
        if c + 1 < N_CH:
            if write_handles[c + 1 - NBUF] is not None:
                write_handles[c + 1 - NBUF].wait()
            build_chunk(c + 1, nb)
        drain_gathers(b, b1, t0)
        if c + 1 < N_CH:
            fire_gathers(nb)
        write_handles[c] = pltpu.async_copy(
            rows_v.at[b],
            out_hbm.at[b1, pl.ds(t0, CH_T)],
            wsem,
        )
    write_handles[N_CH - 2].wait()
    write_handles[N_CH - 1].wait()


def kernel(tokens, table):
    return _emb_lookup(tokens.astype(jnp.int32), table)


# speedup vs baseline: 3.7664x; 1.0295x over previous
import functools

import jax
import jax.numpy as jnp
from jax import lax
from jax.experimental import pallas as pl
from jax.experimental.pallas import tpu as pltpu, tpu_sc as plsc

NUM_CHARS = 1000
D = 16
L_TOK = 20
ROWS_TOK = 22
B1, B2 = 64, 256
NT = B1 * B2
NC, NS, LANES = 2, 16, 16
NW = NC * NS
B1_PER_W = B1 // NW
T_PER_W = NT // NW
CH_T = 128
CH_R = CH_T * ROWS_TOK
N_CH = T_PER_W // CH_T
NBUF = 2
FIRE_U = 4

_mesh = plsc.VectorSubcoreMesh(core_axis_name="c", subcore_axis_name="s")


@functools.partial(
    pl.kernel,
    out_type=jax.ShapeDtypeStruct((B1, B2, ROWS_TOK, D), jnp.float32),
    mesh=_mesh,
    compiler_params=pltpu.CompilerParams(
        use_tc_tiling_on_sc=False, needs_layout_passes=False),
    scratch_types=[
        pltpu.VMEM((T_PER_W * L_TOK,), jnp.int32),
        pltpu.VMEM((NBUF, CH_T, ROWS_TOK), jnp.int32),
        pltpu.VMEM((NBUF, CH_T, ROWS_TOK, D), jnp.float32),
        pltpu.SemaphoreType.DMA,
        pltpu.SemaphoreType.DMA,
    ],
)
def _emb_lookup(tok_hbm, table_hbm, out_hbm, tok_v, fidx_v, rows_v, gsem, wsem):
    w = lax.axis_index("s") * NC + lax.axis_index("c")
    b1_base = w * B1_PER_W
    tok_base = pl.multiple_of(w * (T_PER_W * L_TOK), T_PER_W * L_TOK)
    pltpu.sync_copy(tok_hbm.at[pl.ds(tok_base, T_PER_W * L_TOK)], tok_v)

    c22 = jnp.full((LANES,), ROWS_TOK, jnp.int32)
    c20 = jnp.full((LANES,), L_TOK, jnp.int32)
    c1 = jnp.full((LANES,), 1, jnp.int32)
    c0 = jnp.full((LANES,), 0, jnp.int32)
    c21 = jnp.full((LANES,), ROWS_TOK - 1, jnp.int32)
    cmax = jnp.full((LANES,), T_PER_W * L_TOK - 1, jnp.int32)
    cstart = jnp.full((LANES,), NUM_CHARS, jnp.int32)
    cend = jnp.full((LANES,), NUM_CHARS + 1, jnp.int32)
    lanes_iota = lax.iota(jnp.int32, LANES)

    UNROLL = 4

    def build_chunk(c, b):
        base_r = c * CH_R
        ctok0 = jnp.full((LANES,), c * CH_T, jnp.int32)
        fidx_b = fidx_v.at[b]

        def build(v, carry2):
            for u in range(UNROLL):
                base = jnp.full(
                    (LANES,), base_r + (v * UNROLL + u) * LANES, jnp.int32)
                r = base + lanes_iota
                q = lax.div(r, c22)
                l = r - q * c22
                src = q * c20 + l - c1
                src = jnp.minimum(jnp.maximum(src, c0), cmax)
                tv = plsc.load_gather(tok_v, [src])
                tv = jnp.where(l == c0, cstart, tv)
                tv = jnp.where(l == c21, cend, tv)
                plsc.store_scatter(fidx_b, [q - ctok0, l], tv)
            return carry2

        lax.fori_loop(0, CH_R // (LANES * UNROLL), build, 0)

    def fire_gathers(b):
        def fire(g, carry):
            for u in range(FIRE_U):
                t = g * FIRE_U + u
                pltpu.async_copy(
                    table_hbm.at[fidx_v.at[b, t]],
                    rows_v.at[b, t],
                    gsem,
                )
            return carry

        lax.fori_loop(0, CH_T // FIRE_U, fire, 0)

    def drain_gathers(b, b1, t0):
        pltpu.make_async_copy(
            out_hbm.at[b1, pl.ds(t0, CH_T)], rows_v.at[b], gsem).wait()

    write_handles = [None] * N_CH
    build_chunk(0, 0)
    fire_gathers(0)
    for c in range(N_CH):
        b = c % NBUF
        nb = (c + 1) % NBUF
        b1 = b1_base + c // 2
        t0 = (c % 2) * CH_T
        if c + 1 < N_CH:
            if write_handles[c + 1 - NBUF] is not None:
                write_handles[c + 1 - NBUF].wait()
            build_chunk(c + 1, nb)
        drain_gathers(b, b1, t0)
        if c + 1 < N_CH:
            fire_gathers(nb)
        write_handles[c] = pltpu.async_copy(
            rows_v.at[b],
            out_hbm.at[b1, pl.ds(t0, CH_T)],
            wsem,
        )
    write_handles[N_CH - 2].wait()
    write_handles[N_CH - 1].wait()


def kernel(tokens, table):
    B1_, B2_, L_ = tokens.shape
    return _emb_lookup(tokens.reshape(B1_ * B2_ * L_).astype(jnp.int32), table)


# speedup vs baseline: 12.3893x; 3.2894x over previous
import functools

import jax
import jax.numpy as jnp
from jax import lax
from jax.experimental import pallas as pl
from jax.experimental.pallas import tpu as pltpu, tpu_sc as plsc

NUM_CHARS = 1000
D = 16
L_TOK = 20
ROWS_TOK = 22
B1, B2 = 64, 256
NT = B1 * B2
NC, NS, LANES = 2, 16, 16
NW = NC * NS
B1_PER_W = B1 // NW
TOKW = B1_PER_W * B2 * L_TOK
NBUF = 4

_mesh = plsc.VectorSubcoreMesh(core_axis_name="c", subcore_axis_name="s")


@functools.partial(
    pl.kernel,
    out_type=jax.ShapeDtypeStruct((B1, ROWS_TOK, 2, 2, 8, 128), jnp.float32),
    mesh=_mesh,
    compiler_params=pltpu.CompilerParams(
        use_tc_tiling_on_sc=False, needs_layout_passes=False),
    scratch_types=[
        pltpu.VMEM((NUM_CHARS + 2, D), jnp.float32),
        pltpu.VMEM((TOKW,), jnp.int32),
        pltpu.VMEM((NBUF, 2, 2, 8, 128), jnp.float32),
        pltpu.VMEM((2, 2, 2, 8, 128), jnp.float32),
        pltpu.SemaphoreType.DMA,
    ],
)
def _emb_lookup(tok_hbm, table_hbm, out_hbm, table_v, tok_v, pbuf, sbuf, wsem):
    w = lax.axis_index("s") * NC + lax.axis_index("c")
    tok_base = pl.multiple_of(w * TOKW, TOKW)
    pltpu.sync_copy(table_hbm, table_v)
    pltpu.sync_copy(tok_hbm.at[pl.ds(tok_base, TOKW)], tok_v)

    lanes20 = lax.iota(jnp.int32, LANES) * jnp.full((LANES,), L_TOK, jnp.int32)

    for s in range(2):
        sidx = jnp.full((LANES,), NUM_CHARS + s, jnp.int32)
        for d in range(D):
            ti, sd = d // 8, d % 8
            v = plsc.load_gather(table_v, [sidx, jnp.full((LANES,), d, jnp.int32)])
            for tj in range(2):
                for g8 in range(8):
                    sbuf[s, ti, tj, sd, pl.ds(g8 * 16, 16)] = v

    def write_plane(src, b1, r):
        return pltpu.async_copy(src, out_hbm.at[b1, r], wsem)

    def drain_one_plane():
        pltpu.make_async_copy(out_hbm.at[0, 0], pbuf.at[0], wsem).wait()

    n_planes = 0
    for b1l in range(B1_PER_W):
        b1 = w * B1_PER_W + b1l
        write_plane(sbuf.at[0], b1, 0)
        write_plane(sbuf.at[1], b1, ROWS_TOK - 1)
        n_planes += 2

    for b1l in range(B1_PER_W):
        b1 = w * B1_PER_W + b1l
        tok0 = b1l * (B2 * L_TOK)

        def body(r, carry):
            p = r - 1
            b = lax.rem(r, NBUF)

            @pl.when((r + b1l * L_TOK) >= NBUF + 1)
            def _():
                drain_one_plane()

            for g in range(16):
                cidx = plsc.load_gather(
                    tok_v,
                    [lanes20
                     + jnp.full((LANES,), tok0 + g * 16 * L_TOK + p, jnp.int32)],
                )
                tj, g8 = g // 8, g % 8
                for d in range(D):
                    ti, sd = d // 8, d % 8
                    v = plsc.load_gather(
                        table_v, [cidx, jnp.full((LANES,), d, jnp.int32)])
                    pbuf[b, ti, tj, sd, pl.ds(g8 * 16, 16)] = v
            write_plane(pbuf.at[b], b1, r)
            return carry

        lax.fori_loop(1, ROWS_TOK - 1, body, 0)
        n_planes += L_TOK

    for _ in range(n_planes - 36):
        drain_one_plane()


def kernel(tokens, table):
    tb1, tb2, tl = tokens.shape
    tok_flat = tokens.reshape(tb1 * tb2 * tl).astype(jnp.int32)
    out6 = _emb_lookup(tok_flat, table)
    return out6.transpose(0, 3, 5, 1, 2, 4).reshape(B1, B2, ROWS_TOK, D)


# speedup vs baseline: 21.5422x; 1.7388x over previous
import functools

import jax
import jax.numpy as jnp
from jax import lax
from jax.experimental import pallas as pl
from jax.experimental.pallas import tpu as pltpu, tpu_sc as plsc

NUM_CHARS = 1000
D = 16
L_TOK = 20
ROWS_TOK = 22
B1, B2 = 64, 256
NC, NS, LANES = 2, 16, 16
NW = NC * NS
B1_PER_W = B1 // NW
NBUF = 4

_mesh = plsc.VectorSubcoreMesh(core_axis_name="c", subcore_axis_name="s")


@functools.partial(
    pl.kernel,
    out_type=jax.ShapeDtypeStruct((B1, ROWS_TOK, 2, 2, 8, 128), jnp.float32),
    mesh=_mesh,
    compiler_params=pltpu.CompilerParams(
        use_tc_tiling_on_sc=False, needs_layout_passes=False),
    scratch_types=[
        pltpu.VMEM((D, NUM_CHARS + 2), jnp.float32),
        pltpu.VMEM((L_TOK, 2, 2, 128), jnp.int32),
        pltpu.VMEM((NBUF, 2, 2, 8, 128), jnp.float32),
        pltpu.VMEM((2, 2, 2, 8, 128), jnp.float32),
        pltpu.SemaphoreType.DMA,
    ],
)
def _emb_lookup(tok_hbm, tableT_hbm, out_hbm, table_v, tok_v, pbuf, sbuf, wsem):
    w = lax.axis_index("s") * NC + lax.axis_index("c")
    tb1 = lax.div(w, 4)
    sb0 = lax.rem(w, 4) * 2
    pltpu.sync_copy(tableT_hbm, table_v)
    pltpu.sync_copy(
        tok_hbm.at[pl.ds(0, L_TOK), tb1, pl.ds(0, 2), pl.ds(sb0, 2)], tok_v)

    for s in range(2):
        sidx = jnp.full((LANES,), NUM_CHARS + s, jnp.int32)
        for d in range(D):
            ti, sd = d // 8, d % 8
            v = plsc.load_gather(
                table_v, [jnp.full((LANES,), d, jnp.int32), sidx])
            for tj in range(2):
                for g8 in range(8):
                    sbuf[s, ti, tj, sd, pl.ds(g8 * 16, 16)] = v

    def write_plane(src, b1, r):
        return pltpu.async_copy(src, out_hbm.at[b1, r], wsem)

    def drain_one_plane():
        pltpu.make_async_copy(out_hbm.at[0, 0], pbuf.at[0], wsem).wait()

    n_planes = 0
    for b1l in range(B1_PER_W):
        b1 = w * B1_PER_W + b1l
        write_plane(sbuf.at[0], b1, 0)
        write_plane(sbuf.at[1], b1, ROWS_TOK - 1)
        n_planes += 2

    for b1l in range(B1_PER_W):
        b1 = w * B1_PER_W + b1l

        def body(r, carry):
            p = r - 1
            b = lax.rem(r, NBUF)

            @pl.when((r + b1l * L_TOK) >= NBUF + 1)
            def _():
                drain_one_plane()

            for g in range(16):
                tj, g8 = g // 8, g % 8
                cidx = tok_v[p, tj, b1l, pl.ds(g8 * 16, 16)]
                for d in range(D):
                    ti, sd = d // 8, d % 8
                    v = plsc.load_gather(
                        table_v, [jnp.full((LANES,), d, jnp.int32), cidx])
                    pbuf[b, ti, tj, sd, pl.ds(g8 * 16, 16)] = v
            write_plane(pbuf.at[b], b1, r)
            return carry

        lax.fori_loop(1, ROWS_TOK - 1, body, 0)
        n_planes += L_TOK

    for _ in range(n_planes - 36):
        drain_one_plane()


def kernel(tokens, table):
    tok6 = (tokens.astype(jnp.int32)
            .transpose(2, 0, 1).reshape(L_TOK, 8, 8, 2, 128)
            .transpose(0, 1, 3, 2, 4))
    out6 = _emb_lookup(tok6, table.T)
    return out6.transpose(0, 3, 5, 1, 2, 4).reshape(B1, B2, ROWS_TOK, D)
